# Initial kernel scaffold; baseline (speedup 1.0000x reference)
import jax
import jax.numpy as jnp
from jax.experimental import pallas as pl


def kernel(x, edge_index, edge_attr, batch, x_emb1, x_emb2, gcn_weight, gcn_bias, edge_emb1, edge_emb2, bn_gamma, bn_beta, feat_w, feat_b, out_w1, out_b1, out_w2, out_b2):
    raise NotImplementedError("write your pallas kernel here")


# speedup vs baseline: 4.2287x; 4.2287x over previous
import functools

import jax
import jax.numpy as jnp
from jax import lax
from jax.experimental import pallas as pl
from jax.experimental.pallas import tpu as pltpu
from jax.experimental.pallas import tpu_sc as plsc

N = 10000
E = 320000
D = 128
L = 5
FEAT = 256
G = 256

NC = 2
NS = 16
NW = NC * NS

CH = 128
ET = E + N
NCH = -(-ET // (NW * CH))
E_PAD = NW * NCH * CH
NP = 10112
STRIPE = NP // NS

_mesh = plsc.VectorSubcoreMesh(core_axis_name="c", subcore_axis_name="s")


@functools.partial(
    pl.kernel,
    mesh=_mesh,
    out_type=jax.ShapeDtypeStruct((NC, NP, D), jnp.float32),
    scratch_types=[
        pltpu.VMEM((NCH, CH), jnp.int32),
        pltpu.VMEM((NCH, CH), jnp.int32),
        pltpu.VMEM((CH, D), jnp.float32),
        pltpu.SemaphoreType.DMA,
        pltpu.VMEM_SHARED((NP, D), jnp.float32),
    ],
)
def _spmm_sc(hw_hbm, row_hbm, col_hbm, zero_hbm, out_hbm, ridx, cidx, buf,
             sem, accum):
    c = lax.axis_index("c")
    s = lax.axis_index("s")
    w = s * NC + c
    pltpu.sync_copy(zero_hbm, accum.at[pl.ds(s * STRIPE, STRIPE)])
    pltpu.sync_copy(row_hbm.at[w], ridx)
    pltpu.sync_copy(col_hbm.at[w], cidx)
    plsc.subcore_barrier()

    def step(j, carry):
        pltpu.async_copy(hw_hbm.at[ridx.at[j]], buf, sem).wait()
        pltpu.sync_copy(buf, accum.at[cidx.at[j]], add=True)
        return carry

    lax.fori_loop(0, NCH, step, 0)
    plsc.subcore_barrier()
    pltpu.sync_copy(accum.at[pl.ds(s * STRIPE, STRIPE)],
                    out_hbm.at[c, pl.ds(s * STRIPE, STRIPE)])


def _sall_body(cntp_ref, ecatt_ref, sall_ref):
    cnt = cntp_ref[0] + cntp_ref[1]
    sall_ref[...] = jnp.dot(cnt, ecatt_ref[...],
                            preferred_element_type=jnp.float32,
                            precision=lax.Precision.HIGHEST)


def _emb_body(x0_ref, x1_ref, emb1_ref, emb2_ref, w0_ref, hw_ref):
    cols = lax.broadcasted_iota(jnp.int32, (NP, 128), 1)
    oh1 = (x0_ref[...] == cols).astype(jnp.float32)
    oh2 = (x1_ref[...] == cols).astype(jnp.float32)
    w0 = w0_ref[...]
    emb1w = jnp.dot(emb1_ref[...], w0, preferred_element_type=jnp.float32)
    emb2w = jnp.dot(emb2_ref[...], w0, preferred_element_type=jnp.float32)
    hw = jnp.dot(oh1, emb1w, preferred_element_type=jnp.float32,
                 precision=lax.Precision.HIGHEST)
    hw += jnp.dot(oh2, emb2w, preferred_element_type=jnp.float32,
                  precision=lax.Precision.HIGHEST)
    rows = lax.broadcasted_iota(jnp.int32, (NP, 1), 0)
    hw_ref[...] = jnp.where(rows < N, hw, 0.0)


def _merge_body(p_ref, sall_ref, gamma_ref, beta_ref,
                wnext_ref, out_ref, *, last, layer):
    t = p_ref[0] + p_ref[1]
    svec = sall_ref[:, layer:layer + 1]
    t = t + svec
    m0 = jnp.sum(t, axis=0, keepdims=True) * (1.0 / N)
    dmat = t - m0
    var = (jnp.sum(dmat * dmat, axis=0, keepdims=True)
           - float(NP - N) * m0 * m0) * (1.0 / N)
    inv = lax.rsqrt(var + 1e-5)
    y = dmat * (inv * gamma_ref[...]) + beta_ref[...]
    if not last:
        y = jnp.maximum(y, 0.0)
    rows = lax.broadcasted_iota(jnp.int32, (NP, 1), 0)
    y = jnp.where(rows < N, y, 0.0)
    if wnext_ref is None:
        out_ref[...] = y
    else:
        out_ref[...] = jnp.dot(y, wnext_ref[...],
                               preferred_element_type=jnp.float32)


def _make_merge_mid(layer):
    def _merge_mid(p_ref, sall_ref, gamma_ref, beta_ref, wnext_ref, out_ref):
        _merge_body(p_ref, sall_ref, gamma_ref, beta_ref,
                    wnext_ref, out_ref, last=False, layer=layer)
    return _merge_mid


def _make_merge_last(layer):
    def _merge_last(p_ref, sall_ref, gamma_ref, beta_ref, out_ref):
        _merge_body(p_ref, sall_ref, gamma_ref, beta_ref,
                    None, out_ref, last=True, layer=layer)
    return _merge_last


def _pool_body(h_ref, batch_ref, fw_ref, fb_ref, w1_ref, b1_ref, w2_ref,
               b2_ref, hf_ref, o_ref):
    gids = lax.broadcasted_iota(jnp.int32, (G, NP), 0)
    oh = (batch_ref[...] == gids).astype(jnp.float32)
    sums = jnp.dot(oh, h_ref[...], preferred_element_type=jnp.float32,
                   precision=lax.Precision.HIGHEST)
    cnt = jnp.sum(oh, axis=1, keepdims=True)
    hp = sums / jnp.maximum(cnt, 1.0)
    hf = jnp.dot(hp, fw_ref[...],
                 preferred_element_type=jnp.float32) + fb_ref[...]
    hf_ref[...] = hf
    o1 = jnp.maximum(jnp.dot(hf, w1_ref[...],
                             preferred_element_type=jnp.float32)
                     + b1_ref[...], 0.0)
    o_ref[...] = jnp.dot(o1, w2_ref[...],
                         preferred_element_type=jnp.float32) + b2_ref[...]


def _tc_call(body, out_shapes):
    return pl.pallas_call(body, out_shape=out_shapes)


def kernel(x, edge_index, edge_attr, batch, x_emb1, x_emb2, gcn_weight,
           gcn_bias, edge_emb1, edge_emb2, bn_gamma, bn_beta, feat_w, feat_b,
           out_w1, out_b1, out_w2, out_b2):
    i32 = jnp.int32
    f32 = jnp.float32

    loops = jnp.arange(N, dtype=i32)
    pad_e = E_PAD - ET
    row = jnp.concatenate([edge_index[0].astype(i32), loops,
                           jnp.full((pad_e,), N, i32)])
    col = jnp.concatenate([edge_index[1].astype(i32), loops,
                           jnp.full((pad_e,), N, i32)])
    eidx = jnp.concatenate([
        edge_attr[:, 0].astype(i32) * 3 + edge_attr[:, 1].astype(i32),
        jnp.full((N,), 12, i32),
        jnp.full((pad_e,), 15, i32),
    ])
    row3 = row.reshape(NW, NCH, CH)
    col3 = col.reshape(NW, NCH, CH)
    eidx3 = eidx.reshape(NW, NCH, CH)
    zero_d = jnp.zeros((STRIPE, D), f32)
    ii = jnp.arange(15, dtype=i32)
    ccol = jnp.arange(128, dtype=i32)
    t12 = ((ccol[None, :] == (ii // 3)[:, None]).astype(f32)
           + (ccol[None, :] == (5 + ii % 3)[:, None]).astype(f32))
    t12 = jnp.zeros((NP, D), f32).at[:15].set(t12)

    emb1p = jnp.zeros((128, D), f32).at[:119].set(x_emb1)
    emb2p = jnp.zeros((128, D), f32).at[:3].set(x_emb2)
    x0 = jnp.zeros((NP, 1), i32).at[:N, 0].set(x[:, 0].astype(i32))
    x1 = jnp.zeros((NP, 1), i32).at[:N, 0].set(x[:, 1].astype(i32))
    ecat = jnp.concatenate([edge_emb1[:, :, 0], edge_emb2[:, :, 0]],
                           axis=1)
    ecat_t = jnp.zeros((128, 8), f32).at[:8, :L].set(ecat.T)
    batch_p = jnp.full((1, NP), G, i32).at[0, :N].set(batch.astype(i32))

    cntp = _spmm_sc(t12, eidx3, col3, zero_d)

    s_all = _tc_call(_sall_body, jax.ShapeDtypeStruct((NP, 8), f32))(
        cntp, ecat_t)
    hw = _tc_call(_emb_body, jax.ShapeDtypeStruct((NP, D), f32))(
        x0, x1, emb1p, emb2p, gcn_weight[0])

    h_last = None
    for l in range(L):
        partials = _spmm_sc(hw, row3, col3, zero_d)
        gamma_l = bn_gamma[l][None, :]
        beta_l = bn_beta[l][None, :]
        if l < L - 1:
            hw = _tc_call(_make_merge_mid(l),
                          jax.ShapeDtypeStruct((NP, D), f32))(
                partials, s_all, gamma_l, beta_l, gcn_weight[l + 1])
        else:
            h_last = _tc_call(_make_merge_last(l),
                              jax.ShapeDtypeStruct((NP, D), f32))(
                partials, s_all, gamma_l, beta_l)

    hf, o = _tc_call(_pool_body, (jax.ShapeDtypeStruct((G, FEAT), f32),
                                  jax.ShapeDtypeStruct((G, FEAT // 2), f32)))(
        h_last, batch_p, feat_w.T, feat_b[None, :], out_w1.T,
        out_b1[None, :], out_w2.T, out_b2[None, :])
    return (hf, o)


# speedup vs baseline: 5.4025x; 1.2776x over previous
import functools

import jax
import jax.numpy as jnp
from jax import lax
from jax.experimental import pallas as pl
from jax.experimental.pallas import tpu as pltpu
from jax.experimental.pallas import tpu_sc as plsc

N = 10000
E = 320000
D = 128
L = 5
FEAT = 256
G = 256

NC = 2
NS = 16
NW = NC * NS

CH = 128
ET = E + N
NCHH = 41
NCH = 2 * NCHH
E_PAD = NW * NCH * CH
NP = 10112
STRIPE = NP // NS

_mesh = plsc.VectorSubcoreMesh(core_axis_name="c", subcore_axis_name="s")


def _make_spmm(nt):
    @functools.partial(
        pl.kernel,
        mesh=_mesh,
        out_type=jax.ShapeDtypeStruct((NC, NP, D), jnp.float32),
        scratch_types=[
            pltpu.VMEM((NCHH, CH), jnp.int32),
            pltpu.VMEM((NCHH, CH), jnp.int32),
            pltpu.VMEM((CH, D), jnp.float32),
            pltpu.VMEM((CH, D), jnp.float32),
            pltpu.SemaphoreType.DMA,
            pltpu.SemaphoreType.DMA,
            pltpu.VMEM_SHARED((NP, D), jnp.float32),
        ],
    )
    def _spmm_sc(hw_hbm, row_hbm, col_hbm, zero_hbm, out_hbm, ridx, cidx,
                 buf0, buf1, sem0, sem1, accum):
        c = lax.axis_index("c")
        s = lax.axis_index("s")
        w = s * NC + c
        pltpu.sync_copy(row_hbm.at[w, 0], ridx)
        pltpu.sync_copy(col_hbm.at[w, 0], cidx)
        pltpu.async_copy(hw_hbm.at[ridx.at[0]], buf0, sem0)
        pltpu.sync_copy(zero_hbm, accum.at[pl.ds(s * STRIPE, STRIPE)])
        plsc.subcore_barrier()

        def step(i, carry):
            j0 = 2 * i
            pltpu.async_copy(hw_hbm.at[ridx.at[j0 + 1]], buf1, sem1)
            pltpu.make_async_copy(hw_hbm.at[ridx.at[j0]], buf0, sem0).wait()
            pltpu.sync_copy(buf0, accum.at[cidx.at[j0]], add=True)

            @pl.when(j0 + 2 < NCHH)
            def _():
                pltpu.async_copy(hw_hbm.at[ridx.at[j0 + 2]], buf0, sem0)

            pltpu.make_async_copy(hw_hbm.at[ridx.at[j0 + 1]], buf1,
                                  sem1).wait()
            pltpu.sync_copy(buf1, accum.at[cidx.at[j0 + 1]], add=True)
            return carry

        for h in range(2):
            if h == 1:
                pltpu.sync_copy(row_hbm.at[w, 1], ridx)
                pltpu.sync_copy(col_hbm.at[w, 1], cidx)
                pltpu.async_copy(hw_hbm.at[ridx.at[0]], buf0, sem0)
            lax.fori_loop(0, NCHH // 2, step, 0)
            j = NCHH - 1
            pltpu.make_async_copy(hw_hbm.at[ridx.at[j]], buf0, sem0).wait()
            pltpu.sync_copy(buf0, accum.at[cidx.at[j]], add=True)
        plsc.subcore_barrier()
        pltpu.sync_copy(accum.at[pl.ds(s * STRIPE, STRIPE)],
                        out_hbm.at[c, pl.ds(s * STRIPE, STRIPE)])

    return _spmm_sc


_spmm_sc = _make_spmm(NP)
_spmm_hist = _spmm_sc
NT_HIST = NP


def _sall_body(cntp_ref, ecatt_ref, sall_ref):
    cnt = cntp_ref[0] + cntp_ref[1]
    sall_ref[...] = jnp.dot(cnt, ecatt_ref[...],
                            preferred_element_type=jnp.float32,
                            precision=lax.Precision.HIGHEST)


def _emb_body(x0_ref, x1_ref, emb1_ref, emb2_ref, w0_ref, hw_ref):
    cols = lax.broadcasted_iota(jnp.int32, (NP, 128), 1)
    oh1 = (x0_ref[...] == cols).astype(jnp.float32)
    oh2 = (x1_ref[...] == cols).astype(jnp.float32)
    w0 = w0_ref[...]
    emb1w = jnp.dot(emb1_ref[...], w0, preferred_element_type=jnp.float32)
    emb2w = jnp.dot(emb2_ref[...], w0, preferred_element_type=jnp.float32)
    hw = jnp.dot(oh1, emb1w, preferred_element_type=jnp.float32,
                 precision=lax.Precision.HIGHEST)
    hw += jnp.dot(oh2, emb2w, preferred_element_type=jnp.float32,
                  precision=lax.Precision.HIGHEST)
    rows = lax.broadcasted_iota(jnp.int32, (NP, 1), 0)
    hw_ref[...] = jnp.where(rows < N, hw, 0.0)


def _merge_body(p_ref, sall_ref, gamma_ref, beta_ref,
                wnext_ref, out_ref, *, last, layer):
    t = p_ref[0] + p_ref[1]
    svec = sall_ref[:, layer:layer + 1]
    t = t + svec
    m0 = jnp.sum(t, axis=0, keepdims=True) * (1.0 / N)
    dmat = t - m0
    var = (jnp.sum(dmat * dmat, axis=0, keepdims=True)
           - float(NP - N) * m0 * m0) * (1.0 / N)
    inv = lax.rsqrt(var + 1e-5)
    y = dmat * (inv * gamma_ref[...]) + beta_ref[...]
    if not last:
        y = jnp.maximum(y, 0.0)
    rows = lax.broadcasted_iota(jnp.int32, (NP, 1), 0)
    y = jnp.where(rows < N, y, 0.0)
    if wnext_ref is None:
        out_ref[...] = y
    else:
        out_ref[...] = jnp.dot(y, wnext_ref[...],
                               preferred_element_type=jnp.float32)


def _make_merge_mid(layer):
    def _merge_mid(p_ref, sall_ref, gamma_ref, beta_ref, wnext_ref, out_ref):
        _merge_body(p_ref, sall_ref, gamma_ref, beta_ref,
                    wnext_ref, out_ref, last=False, layer=layer)
    return _merge_mid


def _make_merge_last(layer):
    def _merge_last(p_ref, sall_ref, gamma_ref, beta_ref, out_ref):
        _merge_body(p_ref, sall_ref, gamma_ref, beta_ref,
                    None, out_ref, last=True, layer=layer)
    return _merge_last


def _pool_body(h_ref, batch_ref, fw_ref, fb_ref, w1_ref, b1_ref, w2_ref,
               b2_ref, hf_ref, o_ref):
    gids = lax.broadcasted_iota(jnp.int32, (G, NP), 0)
    oh = (batch_ref[...] == gids).astype(jnp.float32)
    sums = jnp.dot(oh, h_ref[...], preferred_element_type=jnp.float32,
                   precision=lax.Precision.HIGHEST)
    cnt = jnp.sum(oh, axis=1, keepdims=True)
    hp = sums / jnp.maximum(cnt, 1.0)
    hf = jnp.dot(hp, fw_ref[...],
                 preferred_element_type=jnp.float32) + fb_ref[...]
    hf_ref[...] = hf
    o1 = jnp.maximum(jnp.dot(hf, w1_ref[...],
                             preferred_element_type=jnp.float32)
                     + b1_ref[...], 0.0)
    o_ref[...] = jnp.dot(o1, w2_ref[...],
                         preferred_element_type=jnp.float32) + b2_ref[...]


def _tc_call(body, out_shapes):
    return pl.pallas_call(body, out_shape=out_shapes)


def kernel(x, edge_index, edge_attr, batch, x_emb1, x_emb2, gcn_weight,
           gcn_bias, edge_emb1, edge_emb2, bn_gamma, bn_beta, feat_w, feat_b,
           out_w1, out_b1, out_w2, out_b2):
    i32 = jnp.int32
    f32 = jnp.float32

    loops = jnp.arange(N, dtype=i32)
    pad_e = E_PAD - ET
    row = jnp.concatenate([edge_index[0].astype(i32), loops,
                           jnp.full((pad_e,), N, i32)])
    col = jnp.concatenate([edge_index[1].astype(i32), loops,
                           jnp.full((pad_e,), N, i32)])
    eidx = jnp.concatenate([
        edge_attr[:, 0].astype(i32) * 3 + edge_attr[:, 1].astype(i32),
        jnp.full((N,), 12, i32),
        jnp.full((pad_e,), 15, i32),
    ])
    eidx = eidx * 64 + (jnp.arange(E_PAD, dtype=i32) % 64)
    row3 = row.reshape(NW, 2, NCHH, CH)
    col3 = col.reshape(NW, 2, NCHH, CH)
    eidx3 = eidx.reshape(NW, 2, NCHH, CH)
    zero_d = jnp.zeros((STRIPE, D), f32)
    ii = jnp.arange(15, dtype=i32)
    ccol = jnp.arange(128, dtype=i32)
    t12 = ((ccol[None, :] == (ii // 3)[:, None]).astype(f32)
           + (ccol[None, :] == (5 + ii % 3)[:, None]).astype(f32))
    t12 = jnp.zeros((NT_HIST, D), f32).at[:960].set(
        jnp.repeat(t12, 64, axis=0))

    emb1p = jnp.zeros((128, D), f32).at[:119].set(x_emb1)
    emb2p = jnp.zeros((128, D), f32).at[:3].set(x_emb2)
    x0 = jnp.zeros((NP, 1), i32).at[:N, 0].set(x[:, 0].astype(i32))
    x1 = jnp.zeros((NP, 1), i32).at[:N, 0].set(x[:, 1].astype(i32))
    ecat = jnp.concatenate([edge_emb1[:, :, 0], edge_emb2[:, :, 0]],
                           axis=1)
    ecat_t = jnp.zeros((128, 8), f32).at[:8, :L].set(ecat.T)
    batch_p = jnp.full((1, NP), G, i32).at[0, :N].set(batch.astype(i32))

    cntp = _spmm_hist(t12, eidx3, col3, zero_d)

    s_all = _tc_call(_sall_body, jax.ShapeDtypeStruct((NP, 8), f32))(
        cntp, ecat_t)
    hw = _tc_call(_emb_body, jax.ShapeDtypeStruct((NP, D), f32))(
        x0, x1, emb1p, emb2p, gcn_weight[0])

    h_last = None
    for l in range(L):
        partials = _spmm_sc(hw, row3, col3, zero_d)
        gamma_l = bn_gamma[l][None, :]
        beta_l = bn_beta[l][None, :]
        if l < L - 1:
            hw = _tc_call(_make_merge_mid(l),
                          jax.ShapeDtypeStruct((NP, D), f32))(
                partials, s_all, gamma_l, beta_l, gcn_weight[l + 1])
        else:
            h_last = _tc_call(_make_merge_last(l),
                              jax.ShapeDtypeStruct((NP, D), f32))(
                partials, s_all, gamma_l, beta_l)

    hf, o = _tc_call(_pool_body, (jax.ShapeDtypeStruct((G, FEAT), f32),
                                  jax.ShapeDtypeStruct((G, FEAT // 2), f32)))(
        h_last, batch_p, feat_w.T, feat_b[None, :], out_w1.T,
        out_b1[None, :], out_w2.T, out_b2[None, :])
    return (hf, o)


# speedup vs baseline: 5.8402x; 1.0810x over previous
import functools

import jax
import jax.numpy as jnp
from jax import lax
from jax.experimental import pallas as pl
from jax.experimental.pallas import tpu as pltpu
from jax.experimental.pallas import tpu_sc as plsc

N = 10000
E = 320000
D = 128
L = 5
FEAT = 256
G = 256

NC = 2
NS = 16
NW = NC * NS

CH = 128
ET = E + N
NCH_TOT = 2624
E_PAD = NCH_TOT * CH
CHA = 124
CHB = 40
SLA = CHA // 4
SLB = CHB // 4
NP = 10112
STRIPE = NP // NS

_mesh = plsc.VectorSubcoreMesh(core_axis_name="c", subcore_axis_name="s")


def _make_spmm(nt):
    @functools.partial(
        pl.kernel,
        mesh=_mesh,
        out_type=jax.ShapeDtypeStruct((NC, NP, D), jnp.float32),
        scratch_types=[
            pltpu.VMEM((SLA, CH), jnp.int32),
            pltpu.VMEM((SLA, CH), jnp.int32),
            pltpu.VMEM((CH, D), jnp.float32),
            pltpu.VMEM((CH, D), jnp.float32),
            pltpu.SemaphoreType.DMA,
            pltpu.SemaphoreType.DMA,
            pltpu.VMEM_SHARED((NP, D), jnp.float32),
        ],
    )
    def _spmm_sc(hw_hbm, row_a, col_a, row_b, col_b, zero_hbm, out_hbm,
                 ridx, cidx, buf0, buf1, sem0, sem1, accum):
        c = lax.axis_index("c")
        s = lax.axis_index("s")
        pltpu.sync_copy(zero_hbm, accum.at[pl.ds(s * STRIPE, STRIPE)])
        plsc.subcore_barrier()

        def go(row_h, col_h, slab):
            def step(i, carry):
                j0 = 2 * i
                pltpu.async_copy(hw_hbm.at[ridx.at[j0 + 1]], buf1, sem1)
                pltpu.make_async_copy(hw_hbm.at[ridx.at[j0]], buf0,
                                      sem0).wait()
                pltpu.sync_copy(buf0, accum.at[cidx.at[j0]], add=True)

                @pl.when(j0 + 2 < slab)
                def _():
                    pltpu.async_copy(hw_hbm.at[ridx.at[j0 + 2]], buf0, sem0)

                pltpu.make_async_copy(hw_hbm.at[ridx.at[j0 + 1]], buf1,
                                      sem1).wait()
                pltpu.sync_copy(buf1, accum.at[cidx.at[j0 + 1]], add=True)
                return carry

            for st in range(4):
                pltpu.sync_copy(row_h.at[s, st], ridx.at[pl.ds(0, slab)])
                pltpu.sync_copy(col_h.at[s, st], cidx.at[pl.ds(0, slab)])
                pltpu.async_copy(hw_hbm.at[ridx.at[0]], buf0, sem0)
                lax.fori_loop(0, slab // 2, step, 0)
                if slab % 2:
                    j = slab - 1
                    pltpu.make_async_copy(hw_hbm.at[ridx.at[j]], buf0,
                                          sem0).wait()
                    pltpu.sync_copy(buf0, accum.at[cidx.at[j]], add=True)

        @pl.when(c == 0)
        def _():
            go(row_a, col_a, SLA)

        @pl.when(c == 1)
        def _():
            go(row_b, col_b, SLB)

        plsc.subcore_barrier()
        pltpu.sync_copy(accum.at[pl.ds(s * STRIPE, STRIPE)],
                        out_hbm.at[c, pl.ds(s * STRIPE, STRIPE)])

    return _spmm_sc


_spmm_sc = _make_spmm(NP)
_spmm_hist = _spmm_sc
NT_HIST = NP


def _sall_body(cntp_ref, ecatt_ref, sall_ref):
    cnt = cntp_ref[0] + cntp_ref[1]
    sall_ref[...] = jnp.dot(cnt, ecatt_ref[...],
                            preferred_element_type=jnp.float32,
                            precision=lax.Precision.HIGHEST)


def _emb_body(x0_ref, x1_ref, emb1_ref, emb2_ref, w0_ref, hw_ref):
    cols = lax.broadcasted_iota(jnp.int32, (NP, 128), 1)
    oh1 = (x0_ref[...] == cols).astype(jnp.float32)
    oh2 = (x1_ref[...] == cols).astype(jnp.float32)
    w0 = w0_ref[...]
    emb1w = jnp.dot(emb1_ref[...], w0, preferred_element_type=jnp.float32)
    emb2w = jnp.dot(emb2_ref[...], w0, preferred_element_type=jnp.float32)
    hw = jnp.dot(oh1, emb1w, preferred_element_type=jnp.float32,
                 precision=lax.Precision.HIGHEST)
    hw += jnp.dot(oh2, emb2w, preferred_element_type=jnp.float32,
                  precision=lax.Precision.HIGHEST)
    rows = lax.broadcasted_iota(jnp.int32, (NP, 1), 0)
    hw_ref[...] = jnp.where(rows < N, hw, 0.0)


def _merge_body(p_ref, sall_ref, gamma_ref, beta_ref,
                wnext_ref, out_ref, *, last, layer):
    t = p_ref[0] + p_ref[1]
    svec = sall_ref[:, layer:layer + 1]
    t = t + svec
    m0 = jnp.sum(t, axis=0, keepdims=True) * (1.0 / N)
    dmat = t - m0
    var = (jnp.sum(dmat * dmat, axis=0, keepdims=True)
           - float(NP - N) * m0 * m0) * (1.0 / N)
    inv = lax.rsqrt(var + 1e-5)
    y = dmat * (inv * gamma_ref[...]) + beta_ref[...]
    if not last:
        y = jnp.maximum(y, 0.0)
    rows = lax.broadcasted_iota(jnp.int32, (NP, 1), 0)
    y = jnp.where(rows < N, y, 0.0)
    if wnext_ref is None:
        out_ref[...] = y
    else:
        out_ref[...] = jnp.dot(y, wnext_ref[...],
                               preferred_element_type=jnp.float32)


def _make_merge_mid(layer):
    def _merge_mid(p_ref, sall_ref, gamma_ref, beta_ref, wnext_ref, out_ref):
        _merge_body(p_ref, sall_ref, gamma_ref, beta_ref,
                    wnext_ref, out_ref, last=False, layer=layer)
    return _merge_mid


def _make_merge_last(layer):
    def _merge_last(p_ref, sall_ref, gamma_ref, beta_ref, out_ref):
        _merge_body(p_ref, sall_ref, gamma_ref, beta_ref,
                    None, out_ref, last=True, layer=layer)
    return _merge_last


def _pool_body(h_ref, batch_ref, fw_ref, fb_ref, w1_ref, b1_ref, w2_ref,
               b2_ref, hf_ref, o_ref):
    gids = lax.broadcasted_iota(jnp.int32, (G, NP), 0)
    oh = (batch_ref[...] == gids).astype(jnp.float32)
    sums = jnp.dot(oh, h_ref[...], preferred_element_type=jnp.float32,
                   precision=lax.Precision.HIGHEST)
    cnt = jnp.sum(oh, axis=1, keepdims=True)
    hp = sums / jnp.maximum(cnt, 1.0)
    hf = jnp.dot(hp, fw_ref[...],
                 preferred_element_type=jnp.float32) + fb_ref[...]
    hf_ref[...] = hf
    o1 = jnp.maximum(jnp.dot(hf, w1_ref[...],
                             preferred_element_type=jnp.float32)
                     + b1_ref[...], 0.0)
    o_ref[...] = jnp.dot(o1, w2_ref[...],
                         preferred_element_type=jnp.float32) + b2_ref[...]


def _tc_call(body, out_shapes):
    return pl.pallas_call(body, out_shape=out_shapes)


def kernel(x, edge_index, edge_attr, batch, x_emb1, x_emb2, gcn_weight,
           gcn_bias, edge_emb1, edge_emb2, bn_gamma, bn_beta, feat_w, feat_b,
           out_w1, out_b1, out_w2, out_b2):
    i32 = jnp.int32
    f32 = jnp.float32

    loops = jnp.arange(N, dtype=i32)
    pad_e = E_PAD - ET
    row = jnp.concatenate([edge_index[0].astype(i32), loops,
                           jnp.full((pad_e,), N, i32)])
    col = jnp.concatenate([edge_index[1].astype(i32), loops,
                           jnp.full((pad_e,), N, i32)])
    eidx = jnp.concatenate([
        edge_attr[:, 0].astype(i32) * 3 + edge_attr[:, 1].astype(i32),
        jnp.full((N,), 12, i32),
        jnp.full((pad_e,), 15, i32),
    ])
    eidx = eidx * 64 + (jnp.arange(E_PAD, dtype=i32) % 64)
    na = NS * CHA * CH

    def _split(a):
        return (a[:na].reshape(NS, 4, SLA, CH),
                a[na:].reshape(NS, 4, SLB, CH))

    row_a, row_b = _split(row)
    col_a, col_b = _split(col)
    eidx_a, eidx_b = _split(eidx)
    zero_d = jnp.zeros((STRIPE, D), f32)
    ii = jnp.arange(15, dtype=i32)
    ccol = jnp.arange(128, dtype=i32)
    t12 = ((ccol[None, :] == (ii // 3)[:, None]).astype(f32)
           + (ccol[None, :] == (5 + ii % 3)[:, None]).astype(f32))
    t12 = jnp.zeros((NT_HIST, D), f32).at[:960].set(
        jnp.repeat(t12, 64, axis=0))

    emb1p = jnp.zeros((128, D), f32).at[:119].set(x_emb1)
    emb2p = jnp.zeros((128, D), f32).at[:3].set(x_emb2)
    x0 = jnp.zeros((NP, 1), i32).at[:N, 0].set(x[:, 0].astype(i32))
    x1 = jnp.zeros((NP, 1), i32).at[:N, 0].set(x[:, 1].astype(i32))
    ecat = jnp.concatenate([edge_emb1[:, :, 0], edge_emb2[:, :, 0]],
                           axis=1)
    ecat_t = jnp.zeros((128, 8), f32).at[:8, :L].set(ecat.T)
    batch_p = jnp.full((1, NP), G, i32).at[0, :N].set(batch.astype(i32))

    cntp = _spmm_hist(t12, eidx_a, col_a, eidx_b, col_b, zero_d)

    s_all = _tc_call(_sall_body, jax.ShapeDtypeStruct((NP, 8), f32))(
        cntp, ecat_t)
    hw = _tc_call(_emb_body, jax.ShapeDtypeStruct((NP, D), f32))(
        x0, x1, emb1p, emb2p, gcn_weight[0])

    h_last = None
    for l in range(L):
        partials = _spmm_sc(hw, row_a, col_a, row_b, col_b, zero_d)
        gamma_l = bn_gamma[l][None, :]
        beta_l = bn_beta[l][None, :]
        if l < L - 1:
            hw = _tc_call(_make_merge_mid(l),
                          jax.ShapeDtypeStruct((NP, D), f32))(
                partials, s_all, gamma_l, beta_l, gcn_weight[l + 1])
        else:
            h_last = _tc_call(_make_merge_last(l),
                              jax.ShapeDtypeStruct((NP, D), f32))(
                partials, s_all, gamma_l, beta_l)

    hf, o = _tc_call(_pool_body, (jax.ShapeDtypeStruct((G, FEAT), f32),
                                  jax.ShapeDtypeStruct((G, FEAT // 2), f32)))(
        h_last, batch_p, feat_w.T, feat_b[None, :], out_w1.T,
        out_b1[None, :], out_w2.T, out_b2[None, :])
    return (hf, o)


# speedup vs baseline: 13.5571x; 2.3213x over previous
import functools

import jax
import jax.numpy as jnp
from jax import lax
from jax.experimental import pallas as pl
from jax.experimental.pallas import tpu as pltpu
from jax.experimental.pallas import tpu_sc as plsc

N = 10000
E = 320000
D = 128
L = 5
FEAT = 256
G = 256

NC = 2
NS = 16
NW = NC * NS

CH = 128
ET = E + N
NCHH = 41
NCH = 2 * NCHH
E_PAD = NW * NCH * CH
NP = 10112
STRIPE = NP // NS

_mesh = plsc.VectorSubcoreMesh(core_axis_name="c", subcore_axis_name="s")


def _make_spmm(nt):
    @functools.partial(
        pl.kernel,
        mesh=_mesh,
        out_type=jax.ShapeDtypeStruct((NC, NP, D), jnp.float32),
        scratch_types=[
            pltpu.VMEM((NCHH, CH), jnp.int32),
            pltpu.VMEM((NCHH, CH), jnp.int32),
            pltpu.VMEM((CH, D), jnp.float32),
            pltpu.VMEM((CH, D), jnp.float32),
            pltpu.SemaphoreType.DMA,
            pltpu.SemaphoreType.DMA,
            pltpu.VMEM_SHARED((NP, D), jnp.float32),
        ],
    )
    def _spmm_sc(hw_hbm, row_hbm, col_hbm, zero_hbm, out_hbm,
                 ridx, cidx, buf0, buf1, sem0, sem1, accum):
        c = lax.axis_index("c")
        s = lax.axis_index("s")
        w = s * NC + c
        pltpu.sync_copy(zero_hbm, accum.at[pl.ds(s * STRIPE, STRIPE)])
        plsc.subcore_barrier()

        def step(i, carry):
            j0 = 2 * i
            pltpu.async_copy(hw_hbm.at[ridx.at[j0 + 1]], buf1, sem1)
            pltpu.make_async_copy(hw_hbm.at[ridx.at[j0]], buf0, sem0).wait()
            pltpu.sync_copy(buf0, accum.at[cidx.at[j0]], add=True)

            @pl.when(j0 + 2 < NCHH)
            def _():
                pltpu.async_copy(hw_hbm.at[ridx.at[j0 + 2]], buf0, sem0)

            pltpu.make_async_copy(hw_hbm.at[ridx.at[j0 + 1]], buf1,
                                  sem1).wait()
            pltpu.sync_copy(buf1, accum.at[cidx.at[j0 + 1]], add=True)
            return carry

        for h in range(2):
            pltpu.sync_copy(row_hbm.at[w, h], ridx)
            pltpu.sync_copy(col_hbm.at[w, h], cidx)
            pltpu.async_copy(hw_hbm.at[ridx.at[0]], buf0, sem0)
            lax.fori_loop(0, NCHH // 2, step, 0)
            j = NCHH - 1
            pltpu.make_async_copy(hw_hbm.at[ridx.at[j]], buf0, sem0).wait()
            pltpu.sync_copy(buf0, accum.at[cidx.at[j]], add=True)

        plsc.subcore_barrier()
        pltpu.sync_copy(accum.at[pl.ds(s * STRIPE, STRIPE)],
                        out_hbm.at[c, pl.ds(s * STRIPE, STRIPE)])

    return _spmm_sc


_spmm_sc = _make_spmm(NP)
_spmm_hist = _spmm_sc
NT_HIST = NP


def _sall_body(cntp_ref, ecatt_ref, sall_ref):
    cnt = cntp_ref[0] + cntp_ref[1]
    sall_ref[...] = jnp.dot(cnt, ecatt_ref[...],
                            preferred_element_type=jnp.float32,
                            precision=lax.Precision.HIGHEST)


def _emb_body(x0_ref, x1_ref, emb1_ref, emb2_ref, w0_ref, hw_ref):
    cols = lax.broadcasted_iota(jnp.int32, (NP, 128), 1)
    oh1 = (x0_ref[...] == cols).astype(jnp.float32)
    oh2 = (x1_ref[...] == cols).astype(jnp.float32)
    w0 = w0_ref[...]
    emb1w = jnp.dot(emb1_ref[...], w0, preferred_element_type=jnp.float32)
    emb2w = jnp.dot(emb2_ref[...], w0, preferred_element_type=jnp.float32)
    hw = jnp.dot(oh1, emb1w, preferred_element_type=jnp.float32,
                 precision=lax.Precision.HIGHEST)
    hw += jnp.dot(oh2, emb2w, preferred_element_type=jnp.float32,
                  precision=lax.Precision.HIGHEST)
    rows = lax.broadcasted_iota(jnp.int32, (NP, 1), 0)
    hw_ref[...] = jnp.where(rows < N, hw, 0.0)


def _merge_body(p_ref, sall_ref, gamma_ref, beta_ref,
                wnext_ref, out_ref, *, last, layer):
    t = p_ref[0] + p_ref[1]
    svec = sall_ref[:, layer:layer + 1]
    t = t + svec
    m0 = jnp.sum(t, axis=0, keepdims=True) * (1.0 / N)
    dmat = t - m0
    var = (jnp.sum(dmat * dmat, axis=0, keepdims=True)
           - float(NP - N) * m0 * m0) * (1.0 / N)
    inv = lax.rsqrt(var + 1e-5)
    y = dmat * (inv * gamma_ref[...]) + beta_ref[...]
    if not last:
        y = jnp.maximum(y, 0.0)
    rows = lax.broadcasted_iota(jnp.int32, (NP, 1), 0)
    y = jnp.where(rows < N, y, 0.0)
    if wnext_ref is None:
        out_ref[...] = y
    else:
        out_ref[...] = jnp.dot(y, wnext_ref[...],
                               preferred_element_type=jnp.float32)


def _make_merge_mid(layer):
    def _merge_mid(p_ref, sall_ref, gamma_ref, beta_ref, wnext_ref, out_ref):
        _merge_body(p_ref, sall_ref, gamma_ref, beta_ref,
                    wnext_ref, out_ref, last=False, layer=layer)
    return _merge_mid


def _make_merge_last(layer):
    def _merge_last(p_ref, sall_ref, gamma_ref, beta_ref, out_ref):
        _merge_body(p_ref, sall_ref, gamma_ref, beta_ref,
                    None, out_ref, last=True, layer=layer)
    return _merge_last


def _pool_body(h_ref, batch_ref, fw_ref, fb_ref, w1_ref, b1_ref, w2_ref,
               b2_ref, hf_ref, o_ref):
    gids = lax.broadcasted_iota(jnp.int32, (G, NP), 0)
    oh = (batch_ref[...] == gids).astype(jnp.float32)
    sums = jnp.dot(oh, h_ref[...], preferred_element_type=jnp.float32,
                   precision=lax.Precision.HIGHEST)
    cnt = jnp.sum(oh, axis=1, keepdims=True)
    hp = sums / jnp.maximum(cnt, 1.0)
    hf = jnp.dot(hp, fw_ref[...],
                 preferred_element_type=jnp.float32) + fb_ref[...]
    hf_ref[...] = hf
    o1 = jnp.maximum(jnp.dot(hf, w1_ref[...],
                             preferred_element_type=jnp.float32)
                     + b1_ref[...], 0.0)
    o_ref[...] = jnp.dot(o1, w2_ref[...],
                         preferred_element_type=jnp.float32) + b2_ref[...]


def _tc_call(body, out_shapes):
    return pl.pallas_call(body, out_shape=out_shapes)


def kernel(x, edge_index, edge_attr, batch, x_emb1, x_emb2, gcn_weight,
           gcn_bias, edge_emb1, edge_emb2, bn_gamma, bn_beta, feat_w, feat_b,
           out_w1, out_b1, out_w2, out_b2):
    i32 = jnp.int32
    f32 = jnp.float32

    loops = jnp.arange(N, dtype=i32)
    pad_e = E_PAD - ET
    pad_tgt = N + (jnp.arange(pad_e, dtype=i32) % (NP - N))
    row = jnp.concatenate([edge_index[0].astype(i32), loops, pad_tgt])
    col = jnp.concatenate([edge_index[1].astype(i32), loops, pad_tgt])
    eidx = jnp.concatenate([
        edge_attr[:, 0].astype(i32) * 3 + edge_attr[:, 1].astype(i32),
        jnp.full((N,), 12, i32),
        jnp.full((pad_e,), 15, i32),
    ])
    eidx = eidx * 64 + (jnp.arange(E_PAD, dtype=i32) % 64)
    row4 = row.reshape(NW, 2, NCHH, CH)
    col4 = col.reshape(NW, 2, NCHH, CH)
    eidx4 = eidx.reshape(NW, 2, NCHH, CH)
    zero_d = jnp.zeros((STRIPE, D), f32)
    ii = jnp.arange(15, dtype=i32)
    ccol = jnp.arange(128, dtype=i32)
    t12 = ((ccol[None, :] == (ii // 3)[:, None]).astype(f32)
           + (ccol[None, :] == (5 + ii % 3)[:, None]).astype(f32))
    t12 = jnp.zeros((NT_HIST, D), f32).at[:960].set(
        jnp.repeat(t12, 64, axis=0))

    emb1p = jnp.zeros((128, D), f32).at[:119].set(x_emb1)
    emb2p = jnp.zeros((128, D), f32).at[:3].set(x_emb2)
    x0 = jnp.zeros((NP, 1), i32).at[:N, 0].set(x[:, 0].astype(i32))
    x1 = jnp.zeros((NP, 1), i32).at[:N, 0].set(x[:, 1].astype(i32))
    ecat = jnp.concatenate([edge_emb1[:, :, 0], edge_emb2[:, :, 0]],
                           axis=1)
    ecat_t = jnp.zeros((128, 8), f32).at[:8, :L].set(ecat.T)
    batch_p = jnp.full((1, NP), G, i32).at[0, :N].set(batch.astype(i32))

    cntp = _spmm_hist(t12, eidx4, col4, zero_d)

    s_all = _tc_call(_sall_body, jax.ShapeDtypeStruct((NP, 8), f32))(
        cntp, ecat_t)
    hw = _tc_call(_emb_body, jax.ShapeDtypeStruct((NP, D), f32))(
        x0, x1, emb1p, emb2p, gcn_weight[0])

    h_last = None
    for l in range(L):
        partials = _spmm_sc(hw, row4, col4, zero_d)
        gamma_l = bn_gamma[l][None, :]
        beta_l = bn_beta[l][None, :]
        if l < L - 1:
            hw = _tc_call(_make_merge_mid(l),
                          jax.ShapeDtypeStruct((NP, D), f32))(
                partials, s_all, gamma_l, beta_l, gcn_weight[l + 1])
        else:
            h_last = _tc_call(_make_merge_last(l),
                              jax.ShapeDtypeStruct((NP, D), f32))(
                partials, s_all, gamma_l, beta_l)

    hf, o = _tc_call(_pool_body, (jax.ShapeDtypeStruct((G, FEAT), f32),
                                  jax.ShapeDtypeStruct((G, FEAT // 2), f32)))(
        h_last, batch_p, feat_w.T, feat_b[None, :], out_w1.T,
        out_b1[None, :], out_w2.T, out_b2[None, :])
    return (hf, o)


# speedup vs baseline: 14.3615x; 1.0593x over previous
import functools

import jax
import jax.numpy as jnp
from jax import lax
from jax.experimental import pallas as pl
from jax.experimental.pallas import tpu as pltpu
from jax.experimental.pallas import tpu_sc as plsc

N = 10000
E = 320000
D = 128
L = 5
FEAT = 256
G = 256

NC = 2
NS = 16
NW = NC * NS

CH = 128
ET = E + N
NCHH = 41
NCH = 2 * NCHH
E_PAD = NW * NCH * CH
NP = 10112
STRIPE = NP // NS

_mesh = plsc.VectorSubcoreMesh(core_axis_name="c", subcore_axis_name="s")


def _make_spmm(nt):
    @functools.partial(
        pl.kernel,
        mesh=_mesh,
        out_type=jax.ShapeDtypeStruct((NC, NP, D), jnp.float32),
        scratch_types=[
            pltpu.VMEM((NCHH, CH), jnp.int32),
            pltpu.VMEM((NCHH, CH), jnp.int32),
            pltpu.VMEM((CH, D), jnp.float32),
            pltpu.VMEM((CH, D), jnp.float32),
            pltpu.SemaphoreType.DMA,
            pltpu.SemaphoreType.DMA,
            pltpu.VMEM_SHARED((NP, D), jnp.float32),
        ],
    )
    def _spmm_sc(hw_hbm, row_hbm, col_hbm, zero_hbm, out_hbm,
                 ridx, cidx, buf0, buf1, sem0, sem1, accum):
        c = lax.axis_index("c")
        s = lax.axis_index("s")
        w = s * NC + c
        pltpu.sync_copy(zero_hbm.at[pl.ds(s * STRIPE, STRIPE)],
                        accum.at[pl.ds(s * STRIPE, STRIPE)])
        plsc.subcore_barrier()

        def step(i, carry):
            j0 = 2 * i
            pltpu.async_copy(hw_hbm.at[ridx.at[j0 + 1]], buf1, sem1)
            pltpu.make_async_copy(hw_hbm.at[ridx.at[j0]], buf0, sem0).wait()
            pltpu.sync_copy(buf0, accum.at[cidx.at[j0]], add=True)

            @pl.when(j0 + 2 < NCHH)
            def _():
                pltpu.async_copy(hw_hbm.at[ridx.at[j0 + 2]], buf0, sem0)

            pltpu.make_async_copy(hw_hbm.at[ridx.at[j0 + 1]], buf1,
                                  sem1).wait()
            pltpu.sync_copy(buf1, accum.at[cidx.at[j0 + 1]], add=True)
            return carry

        for h in range(2):
            pltpu.sync_copy(row_hbm.at[w, h], ridx)
            pltpu.sync_copy(col_hbm.at[w, h], cidx)
            pltpu.async_copy(hw_hbm.at[ridx.at[0]], buf0, sem0)
            lax.fori_loop(0, NCHH // 2, step, 0)
            j = NCHH - 1
            pltpu.make_async_copy(hw_hbm.at[ridx.at[j]], buf0, sem0).wait()
            pltpu.sync_copy(buf0, accum.at[cidx.at[j]], add=True)

        plsc.subcore_barrier()
        pltpu.sync_copy(accum.at[pl.ds(s * STRIPE, STRIPE)],
                        out_hbm.at[c, pl.ds(s * STRIPE, STRIPE)])

    return _spmm_sc


_spmm_sc = _make_spmm(NP)
_spmm_hist = _spmm_sc
NT_HIST = NP


def _sall_body(cntp_ref, ecatt_ref, sall_ref):
    cnt = cntp_ref[0] + cntp_ref[1]
    sall_ref[...] = jnp.dot(cnt, ecatt_ref[...],
                            preferred_element_type=jnp.float32,
                            precision=lax.Precision.HIGHEST)


def _emb_body(x0_ref, x1_ref, emb1_ref, emb2_ref, w0_ref, hw_ref):
    cols = lax.broadcasted_iota(jnp.int32, (NP, 128), 1)
    oh1 = (x0_ref[...] == cols).astype(jnp.float32)
    oh2 = (x1_ref[...] == cols).astype(jnp.float32)
    w0 = w0_ref[...]
    emb1w = jnp.dot(emb1_ref[...], w0, preferred_element_type=jnp.float32)
    emb2w = jnp.dot(emb2_ref[...], w0, preferred_element_type=jnp.float32)
    hw = jnp.dot(oh1, emb1w, preferred_element_type=jnp.float32,
                 precision=lax.Precision.HIGHEST)
    hw += jnp.dot(oh2, emb2w, preferred_element_type=jnp.float32,
                  precision=lax.Precision.HIGHEST)
    rows = lax.broadcasted_iota(jnp.int32, (NP, 1), 0)
    hw_ref[...] = jnp.where(rows < N, hw, 0.0)


def _merge_body(p_ref, sall_ref, gamma_ref, beta_ref,
                wnext_ref, out_ref, *, last, layer):
    t = p_ref[0] + p_ref[1]
    svec = sall_ref[:, layer:layer + 1]
    t = t + svec
    m0 = jnp.sum(t, axis=0, keepdims=True) * (1.0 / N)
    dmat = t - m0
    var = (jnp.sum(dmat * dmat, axis=0, keepdims=True)
           - float(NP - N) * m0 * m0) * (1.0 / N)
    inv = lax.rsqrt(var + 1e-5)
    y = dmat * (inv * gamma_ref[...]) + beta_ref[...]
    if not last:
        y = jnp.maximum(y, 0.0)
    rows = lax.broadcasted_iota(jnp.int32, (NP, 1), 0)
    y = jnp.where(rows < N, y, 0.0)
    if wnext_ref is None:
        out_ref[...] = y
    else:
        out_ref[...] = jnp.dot(y, wnext_ref[...],
                               preferred_element_type=jnp.float32)


def _make_merge_mid(layer):
    def _merge_mid(p_ref, sall_ref, gamma_ref, beta_ref, wnext_ref, out_ref):
        _merge_body(p_ref, sall_ref, gamma_ref, beta_ref,
                    wnext_ref, out_ref, last=False, layer=layer)
    return _merge_mid


def _make_merge_last(layer):
    def _merge_last(p_ref, sall_ref, gamma_ref, beta_ref, out_ref):
        _merge_body(p_ref, sall_ref, gamma_ref, beta_ref,
                    None, out_ref, last=True, layer=layer)
    return _merge_last


def _pool_body(h_ref, batch_ref, fw_ref, fb_ref, w1_ref, b1_ref, w2_ref,
               b2_ref, hf_ref, o_ref):
    gids = lax.broadcasted_iota(jnp.int32, (G, NP), 0)
    oh = (batch_ref[...] == gids).astype(jnp.float32)
    sums = jnp.dot(oh, h_ref[...], preferred_element_type=jnp.float32,
                   precision=lax.Precision.HIGHEST)
    cnt = jnp.sum(oh, axis=1, keepdims=True)
    hp = sums / jnp.maximum(cnt, 1.0)
    hf = jnp.dot(hp, fw_ref[...],
                 preferred_element_type=jnp.float32) + fb_ref[...]
    hf_ref[...] = hf
    o1 = jnp.maximum(jnp.dot(hf, w1_ref[...],
                             preferred_element_type=jnp.float32)
                     + b1_ref[...], 0.0)
    o_ref[...] = jnp.dot(o1, w2_ref[...],
                         preferred_element_type=jnp.float32) + b2_ref[...]


def _tc_call(body, out_shapes):
    return pl.pallas_call(body, out_shape=out_shapes)


def kernel(x, edge_index, edge_attr, batch, x_emb1, x_emb2, gcn_weight,
           gcn_bias, edge_emb1, edge_emb2, bn_gamma, bn_beta, feat_w, feat_b,
           out_w1, out_b1, out_w2, out_b2):
    i32 = jnp.int32
    f32 = jnp.float32

    loops = jnp.arange(N, dtype=i32)
    pad_e = E_PAD - ET
    pad_tgt = N + (jnp.arange(pad_e, dtype=i32) % (NP - N))
    row = jnp.concatenate([edge_index[0].astype(i32), loops, pad_tgt])
    col = jnp.concatenate([edge_index[1].astype(i32), loops, pad_tgt])
    eidx = jnp.concatenate([
        edge_attr[:, 0].astype(i32) * 3 + edge_attr[:, 1].astype(i32),
        jnp.full((N,), 12, i32),
        jnp.full((pad_e,), 15, i32),
    ])
    eidx = eidx * 512 + (jnp.arange(E_PAD, dtype=i32) % 512)
    row4 = row.reshape(NW, 2, NCHH, CH)
    col4 = col.reshape(NW, 2, NCHH, CH)
    eidx4 = eidx.reshape(NW, 2, NCHH, CH)
    zero_d = jnp.zeros((NP, D), f32)
    ii = jnp.arange(15, dtype=i32)
    ccol = jnp.arange(128, dtype=i32)
    t12 = ((ccol[None, :] == (ii // 3)[:, None]).astype(f32)
           + (ccol[None, :] == (5 + ii % 3)[:, None]).astype(f32))
    t12 = jnp.zeros((NT_HIST, D), f32).at[:15 * 512].set(
        jnp.repeat(t12, 512, axis=0))

    emb1p = jnp.zeros((128, D), f32).at[:119].set(x_emb1)
    emb2p = jnp.zeros((128, D), f32).at[:3].set(x_emb2)
    x0 = jnp.zeros((NP, 1), i32).at[:N, 0].set(x[:, 0].astype(i32))
    x1 = jnp.zeros((NP, 1), i32).at[:N, 0].set(x[:, 1].astype(i32))
    ecat = jnp.concatenate([edge_emb1[:, :, 0], edge_emb2[:, :, 0]],
                           axis=1)
    ecat_t = jnp.zeros((128, 8), f32).at[:8, :L].set(ecat.T)
    batch_p = jnp.full((1, NP), G, i32).at[0, :N].set(batch.astype(i32))

    cntp = _spmm_hist(t12, eidx4, col4, zero_d)

    s_all = _tc_call(_sall_body, jax.ShapeDtypeStruct((NP, 8), f32))(
        cntp, ecat_t)
    hw = _tc_call(_emb_body, jax.ShapeDtypeStruct((NP, D), f32))(
        x0, x1, emb1p, emb2p, gcn_weight[0])

    h_last = None
    for l in range(L):
        partials = _spmm_sc(hw, row4, col4, zero_d)
        gamma_l = bn_gamma[l][None, :]
        beta_l = bn_beta[l][None, :]
        if l < L - 1:
            hw = _tc_call(_make_merge_mid(l),
                          jax.ShapeDtypeStruct((NP, D), f32))(
                partials, s_all, gamma_l, beta_l, gcn_weight[l + 1])
        else:
            h_last = _tc_call(_make_merge_last(l),
                              jax.ShapeDtypeStruct((NP, D), f32))(
                partials, s_all, gamma_l, beta_l)

    hf, o = _tc_call(_pool_body, (jax.ShapeDtypeStruct((G, FEAT), f32),
                                  jax.ShapeDtypeStruct((G, FEAT // 2), f32)))(
        h_last, batch_p, feat_w.T, feat_b[None, :], out_w1.T,
        out_b1[None, :], out_w2.T, out_b2[None, :])
    return (hf, o)
